# baseline (speedup 1.0000x reference)
import functools

import jax
import jax.numpy as jnp
from jax import lax
from jax.experimental import pallas as pl
from jax.experimental.pallas import tpu as pltpu
from jax.experimental.pallas import tpu_sc as plsc


@functools.lru_cache(maxsize=None)
def _make_gather(batch: int, seq: int, vocab: int, hidden: int):
    info = plsc.get_sparse_core_info()
    num_cores, num_subcores = info.num_cores, info.num_subcores
    num_workers = num_cores * num_subcores
    n_total = batch * seq
    assert n_total % num_workers == 0
    n_per_w = n_total // num_workers
    assert seq % n_per_w == 0
    chunk = 16
    nbuf = 4
    la = 3
    assert n_per_w % chunk == 0
    n_chunks = n_per_w // chunk
    n_groups = n_chunks // nbuf
    assert n_chunks % nbuf == 0 and la < nbuf

    mesh = plsc.VectorSubcoreMesh(core_axis_name="c", subcore_axis_name="s")

    @functools.partial(
        pl.kernel,
        mesh=mesh,
        out_type=jax.ShapeDtypeStruct((batch, seq, hidden), jnp.float32),
        scratch_types=[
            pltpu.VMEM((n_per_w,), jnp.int32),
        ]
        + [pltpu.VMEM((chunk, hidden), jnp.float32) for _ in range(nbuf)]
        + [pltpu.SemaphoreType.DMA for _ in range(2 * nbuf)],
    )
    def gather_kernel(table_hbm, idx_hbm, out_hbm, idx_v, *scratch):
        bufs = scratch[:nbuf]
        gsems = scratch[nbuf : 2 * nbuf]
        ssems = scratch[2 * nbuf :]
        wid = lax.axis_index("s") * num_cores + lax.axis_index("c")
        row = wid // (seq // n_per_w)
        col = (wid % (seq // n_per_w)) * n_per_w
        pltpu.sync_copy(idx_hbm.at[row, pl.ds(col, n_per_w)], idx_v)

        def issue_gather(c, b):
            pltpu.async_copy(
                table_hbm.at[idx_v.at[pl.ds(c * chunk, chunk)]],
                bufs[b], gsems[b],
            )

        def issue_store(d, b):
            pltpu.async_copy(
                bufs[b], out_hbm.at[row, pl.ds(col + d * chunk, chunk)],
                ssems[b],
            )

        def wait_gather(b):
            pltpu.make_async_copy(
                table_hbm.at[idx_v.at[pl.ds(0, chunk)]], bufs[b], gsems[b]
            ).wait()

        def wait_store(b):
            pltpu.make_async_copy(
                bufs[b], out_hbm.at[row, pl.ds(col, chunk)], ssems[b]
            ).wait()

        for c in range(nbuf):
            issue_gather(c, c)
            d = c - la
            if d >= 0:
                wait_gather(d)
                issue_store(d, d)

        @pl.loop(1, n_groups)
        def _grp(grp):
            for b in range(nbuf):
                c = grp * nbuf + b
                wait_store(b)
                issue_gather(c, b)
                bd = (b - la) % nbuf
                wait_gather(bd)
                issue_store(c - la, bd)

        for d in range(n_chunks - la, n_chunks):
            b = d % nbuf
            wait_gather(b)
            issue_store(d, b)
        for b in range(nbuf):
            wait_store(b)

    return gather_kernel


def kernel(input_ids, embed_tokens):
    b, s = input_ids.shape
    vocab, hidden = embed_tokens.shape
    gather = _make_gather(b, s, vocab, hidden)
    return gather(embed_tokens, input_ids.astype(jnp.int32))


# speedup vs baseline: 1.0017x; 1.0017x over previous
import functools

import jax
import jax.numpy as jnp
from jax import lax
from jax.experimental import pallas as pl
from jax.experimental.pallas import tpu as pltpu
from jax.experimental.pallas import tpu_sc as plsc


@functools.lru_cache(maxsize=None)
def _make_gather(batch: int, seq: int, vocab: int, hidden: int):
    info = plsc.get_sparse_core_info()
    num_cores, num_subcores = info.num_cores, info.num_subcores
    num_workers = num_cores * num_subcores
    n_total = batch * seq
    assert n_total % num_workers == 0
    n_per_w = n_total // num_workers
    assert seq % n_per_w == 0
    chunk = 16
    nbuf = 4
    la = 3
    assert n_per_w % chunk == 0
    n_chunks = n_per_w // chunk
    n_groups = n_chunks // nbuf
    assert n_chunks % nbuf == 0 and la < nbuf

    mesh = plsc.VectorSubcoreMesh(core_axis_name="c", subcore_axis_name="s")

    @functools.partial(
        pl.kernel,
        mesh=mesh,
        out_type=jax.ShapeDtypeStruct((batch, seq, hidden), jnp.float32),
        scratch_types=[
            pltpu.VMEM((n_per_w,), jnp.int32),
        ]
        + [pltpu.VMEM((chunk, hidden), jnp.float32) for _ in range(nbuf)]
        + [pltpu.SemaphoreType.DMA for _ in range(2 * nbuf + 2)],
    )
    def gather_kernel(table_hbm, idx_hbm, out_hbm, idx_v, *scratch):
        bufs = scratch[:nbuf]
        gsems = scratch[nbuf : 2 * nbuf]
        ssems = scratch[2 * nbuf : 3 * nbuf]
        isem_h = scratch[3 * nbuf]
        isem_t = scratch[3 * nbuf + 1]
        wid = lax.axis_index("s") * num_cores + lax.axis_index("c")
        row = wid // (seq // n_per_w)
        col = (wid % (seq // n_per_w)) * n_per_w
        head = 128
        h_idx = pltpu.async_copy(
            idx_hbm.at[row, pl.ds(col, head)], idx_v.at[pl.ds(0, head)], isem_h
        )
        t_idx = pltpu.async_copy(
            idx_hbm.at[row, pl.ds(col + head, n_per_w - head)],
            idx_v.at[pl.ds(head, n_per_w - head)],
            isem_t,
        )
        h_idx.wait()

        def issue_gather(c, b):
            pltpu.async_copy(
                table_hbm.at[idx_v.at[pl.ds(c * chunk, chunk)]],
                bufs[b], gsems[b],
            )

        def issue_store(d, b):
            pltpu.async_copy(
                bufs[b], out_hbm.at[row, pl.ds(col + d * chunk, chunk)],
                ssems[b],
            )

        def wait_gather(b):
            pltpu.make_async_copy(
                table_hbm.at[idx_v.at[pl.ds(0, chunk)]], bufs[b], gsems[b]
            ).wait()

        def wait_store(b):
            pltpu.make_async_copy(
                bufs[b], out_hbm.at[row, pl.ds(col, chunk)], ssems[b]
            ).wait()

        for c in range(nbuf):
            issue_gather(c, c)
            d = c - la
            if d >= 0:
                wait_gather(d)
                issue_store(d, d)

        t_idx.wait()

        @pl.loop(1, n_groups)
        def _grp(grp):
            for b in range(nbuf):
                c = grp * nbuf + b
                wait_store(b)
                issue_gather(c, b)
                bd = (b - la) % nbuf
                wait_gather(bd)
                issue_store(c - la, bd)

        for d in range(n_chunks - la, n_chunks):
            b = d % nbuf
            wait_gather(b)
            issue_store(d, b)
        for b in range(nbuf):
            wait_store(b)

    return gather_kernel


def kernel(input_ids, embed_tokens):
    b, s = input_ids.shape
    vocab, hidden = embed_tokens.shape
    gather = _make_gather(b, s, vocab, hidden)
    return gather(embed_tokens, input_ids.astype(jnp.int32))
